# Initial kernel scaffold; baseline (speedup 1.0000x reference)
import jax
import jax.numpy as jnp
from jax.experimental import pallas as pl


def kernel(x, edge_index, node_idx, apart_feature, W1l, b1l, W1r, W2l, b2l, W2r, fW1, fb1, fW2, fb2, fW3, fb3):
    raise NotImplementedError("write your pallas kernel here")


# speedup vs baseline: 5.4035x; 5.4035x over previous
import functools

import jax
import jax.numpy as jnp
from jax import lax
from jax.experimental import pallas as pl
from jax.experimental.pallas import tpu as pltpu
from jax.experimental.pallas import tpu_sc as plsc

_N = 10000
_NP = 10240
_DD = 128
_NC = 2
_NS = 16
_NW = _NC * _NS
_CH = 80
_BM = 1024


def _vec_mesh():
    return plsc.VectorSubcoreMesh(core_axis_name="c", subcore_axis_name="s")


def _make_segsum(E, with_count):
    ew = E // _NW
    nch = ew // _CH
    rp = _NP // _NS

    out_type = [jax.ShapeDtypeStruct((_NC, _NP, _DD), jnp.float32)]
    scratch = [
        pltpu.VMEM((_CH,), jnp.int32),
        pltpu.VMEM((_CH,), jnp.int32),
        pltpu.VMEM((_CH, _DD), jnp.float32),
        pltpu.VMEM_SHARED((_NP, _DD), jnp.float32),
        pltpu.SemaphoreType.DMA,
    ]
    if with_count:
        out_type.append(jax.ShapeDtypeStruct((_NC, _NP), jnp.float32))
        scratch = ([pltpu.VMEM((_CH,), jnp.float32),
                    pltpu.VMEM_SHARED((_NP,), jnp.float32)] + scratch)

    def body(tab, srch, dsth, zrows, zcnt, *rest):
        if with_count:
            (aggo, cnto, onesv, cntsh, srcv, dstv, rows, aggsh, sem) = rest
        else:
            (aggo, srcv, dstv, rows, aggsh, sem) = rest
        c = lax.axis_index("c")
        s = lax.axis_index("s")
        wid = c * _NS + s
        pltpu.sync_copy(zrows.at[pl.ds(s * rp, rp)], aggsh.at[pl.ds(s * rp, rp)])
        if with_count:
            pltpu.sync_copy(zcnt.at[pl.ds(s * rp, rp)], cntsh.at[pl.ds(s * rp, rp)])

            @pl.loop(0, _CH, step=16)
            def _(i):
                onesv[pl.ds(i, 16)] = jnp.ones((16,), jnp.float32)

        plsc.subcore_barrier()

        base0 = wid * ew

        @pl.loop(0, nch)
        def _(k):
            b = base0 + k * _CH
            pltpu.sync_copy(srch.at[pl.ds(b, _CH)], srcv)
            pltpu.sync_copy(dsth.at[pl.ds(b, _CH)], dstv)
            pltpu.async_copy(tab.at[srcv], rows, sem).wait()
            pltpu.sync_copy(rows, aggsh.at[dstv], add=True)
            if with_count:
                pltpu.sync_copy(onesv, cntsh.at[dstv], add=True)

        plsc.subcore_barrier()
        pltpu.sync_copy(aggsh.at[pl.ds(s * rp, rp)],
                        aggo.at[c].at[pl.ds(s * rp, rp)])
        if with_count:
            pltpu.sync_copy(cntsh.at[pl.ds(s * rp, rp)],
                            cnto.at[c].at[pl.ds(s * rp, rp)])

    return pl.kernel(body, out_type=out_type, mesh=_vec_mesh(),
                     scratch_types=scratch)


def _layer1_body(x_ref, agg_ref, cnt_ref, wlT_ref, bl_ref, wrT_ref,
                 h_ref, inv_ref):
    cnt = cnt_ref[:, 0:1] + cnt_ref[:, 1:2]
    inv = 1.0 / jnp.maximum(cnt, 1.0)
    mean = (agg_ref[0] + agg_ref[1]) * inv
    acc = (jnp.dot(mean, wlT_ref[...], preferred_element_type=jnp.float32,
                   precision=lax.Precision.HIGHEST)
           + jnp.dot(x_ref[...], wrT_ref[...], preferred_element_type=jnp.float32,
                     precision=lax.Precision.HIGHEST)
           + bl_ref[...])
    h_ref[...] = jnp.maximum(acc, 0.0)
    inv_ref[...] = jnp.broadcast_to(inv, (inv.shape[0], _DD))


def _layer1_tc(xp, aggp, cnt_t, wlT, bl2, wrT):
    grid = (_NP // _BM,)
    return pl.pallas_call(
        _layer1_body,
        grid=grid,
        in_specs=[
            pl.BlockSpec((_BM, _DD), lambda i: (i, 0)),
            pl.BlockSpec((_NC, _BM, _DD), lambda i: (0, i, 0)),
            pl.BlockSpec((_BM, _NC), lambda i: (i, 0)),
            pl.BlockSpec((_DD, _DD), lambda i: (0, 0)),
            pl.BlockSpec((1, _DD), lambda i: (0, 0)),
            pl.BlockSpec((_DD, _DD), lambda i: (0, 0)),
        ],
        out_specs=[
            pl.BlockSpec((_BM, _DD), lambda i: (i, 0)),
            pl.BlockSpec((_BM, _DD), lambda i: (i, 0)),
        ],
        out_shape=[
            jax.ShapeDtypeStruct((_NP, _DD), jnp.float32),
            jax.ShapeDtypeStruct((_NP, _DD), jnp.float32),
        ],
    )(xp, aggp, cnt_t, wlT, bl2, wrT)


def _make_gather_sel(B):
    bw = B // _NW

    def body(h_hbm, a0_hbm, a1_hbm, inv_hbm, ni_hbm,
             hsel_o, a0_o, a1_o, inv_o, niv, rbuf, ibuf, sem):
        c = lax.axis_index("c")
        s = lax.axis_index("s")
        base = (c * _NS + s) * bw
        pltpu.sync_copy(ni_hbm.at[pl.ds(base, bw)], niv)
        pltpu.async_copy(h_hbm.at[niv], rbuf, sem).wait()
        pltpu.sync_copy(rbuf, hsel_o.at[pl.ds(base, bw)])
        pltpu.async_copy(a0_hbm.at[niv], rbuf, sem).wait()
        pltpu.sync_copy(rbuf, a0_o.at[pl.ds(base, bw)])
        pltpu.async_copy(a1_hbm.at[niv], rbuf, sem).wait()
        pltpu.sync_copy(rbuf, a1_o.at[pl.ds(base, bw)])
        pltpu.async_copy(inv_hbm.at[niv], ibuf, sem).wait()
        pltpu.sync_copy(ibuf, inv_o.at[pl.ds(base, bw)])

    return pl.kernel(
        body,
        out_type=[
            jax.ShapeDtypeStruct((B, _DD), jnp.float32),
            jax.ShapeDtypeStruct((B, _DD), jnp.float32),
            jax.ShapeDtypeStruct((B, _DD), jnp.float32),
            jax.ShapeDtypeStruct((B, _DD), jnp.float32),
        ],
        mesh=_vec_mesh(),
        scratch_types=[
            pltpu.VMEM((bw,), jnp.int32),
            pltpu.VMEM((bw, _DD), jnp.float32),
            pltpu.VMEM((bw, _DD), jnp.float32),
            pltpu.SemaphoreType.DMA,
        ],
    )


def _leaky(v, slope):
    return jnp.where(v > 0, v, slope * v)


def _head_body(hs_ref, a0_ref, a1_ref, inv_ref, ap_ref,
               w2lT_ref, b2l_ref, w2rT_ref, f1aT_ref, f1bT_ref, fb1_ref,
               f2T_ref, fb2_ref, f3T_ref, fb3_ref, out_ref):
    hp = lax.Precision.HIGHEST
    mean2 = (a0_ref[...] + a1_ref[...]) * inv_ref[:, 0:1]
    h2 = (jnp.dot(mean2, w2lT_ref[...], preferred_element_type=jnp.float32,
                  precision=hp)
          + jnp.dot(hs_ref[...], w2rT_ref[...], preferred_element_type=jnp.float32,
                    precision=hp)
          + b2l_ref[...])
    z = (jnp.dot(h2, f1aT_ref[...], preferred_element_type=jnp.float32,
                 precision=hp)
         + jnp.dot(ap_ref[...], f1bT_ref[...], preferred_element_type=jnp.float32,
                   precision=hp)
         + fb1_ref[...])
    z = _leaky(z, 0.1)
    z = jnp.dot(z, f2T_ref[...], preferred_element_type=jnp.float32,
                precision=hp) + fb2_ref[...]
    z = _leaky(z, 0.05)
    z = jnp.dot(z, f3T_ref[...], preferred_element_type=jnp.float32,
                precision=hp) + fb3_ref[...]
    out_ref[...] = z


def _head_tc(B, hsel, a0s, a1s, invs, apart_p, w2lT, b2l2, w2rT,
             f1aT, f1bT, fb12, f2T, fb22, f3T, fb32):
    return pl.pallas_call(
        _head_body,
        out_shape=jax.ShapeDtypeStruct((B, 1), jnp.float32),
    )(hsel, a0s, a1s, invs, apart_p, w2lT, b2l2, w2rT,
      f1aT, f1bT, fb12, f2T, fb22, f3T, fb32)


def kernel(x, edge_index, node_idx, apart_feature, W1l, b1l, W1r,
           W2l, b2l, W2r, fW1, fb1, fW2, fb2, fW3, fb3):
    f32 = jnp.float32
    E = edge_index.shape[1]
    B = node_idx.shape[0]
    AP = apart_feature.shape[1]

    src = edge_index[0]
    dst = edge_index[1]
    xp = jnp.zeros((_NP, _DD), f32).at[:_N].set(x)
    zrows = jnp.zeros((_NP, _DD), f32)
    zcnt = jnp.zeros((_NP,), f32)

    aggp, cntp = _make_segsum(E, True)(xp, src, dst, zrows, zcnt)
    cnt_t = cntp.T

    h, inv16 = _layer1_tc(xp, aggp, cnt_t, W1l.T, b1l.reshape(1, -1), W1r.T)

    agg2p, = _make_segsum(E, False)(h, src, dst, zrows, zcnt)
    a20 = agg2p[0]
    a21 = agg2p[1]

    hsel, a0s, a1s, invs = _make_gather_sel(B)(h, a20, a21, inv16, node_idx)

    apart_p = jnp.pad(apart_feature, ((0, 0), (0, _DD - AP)))
    f1bT = jnp.pad(fW1[:, _DD:], ((0, 0), (0, _DD - AP))).T
    z = _head_tc(B, hsel, a0s, a1s, invs, apart_p,
                 W2l.T, b2l.reshape(1, -1), W2r.T,
                 fW1[:, :_DD].T, f1bT, fb1.reshape(1, -1),
                 fW2.T, fb2.reshape(1, -1), fW3.T, fb3.reshape(1, 1))
    return z
